# baseline (speedup 1.0000x reference)
import jax
import jax.numpy as jnp
from jax.experimental import pallas as pl
from jax.experimental.pallas import tpu as pltpu

SLOT_DIM = 142
NUM_ARCHETYPES = 19
HIDDEN_DIM = 256
CODE_DIM = 64
NUM_CODES = 512
COMMIT_COST = 0.25
B = 16384
BB = 2048


def _dot(a, b):
    return jax.lax.dot_general(a, b, (((1,), (0,)), ((), ())),
                               preferred_element_type=jnp.float32)


def _dot_bf16(p, q):
    return jax.lax.dot_general(p.astype(jnp.bfloat16), q.astype(jnp.bfloat16),
                               (((1,), (0,)), ((), ())),
                               preferred_element_type=jnp.float32)


def _vqvae_body(x_ref, a_ref, w1_ref, b1_ref, w2_ref, b2_ref,
                w3_ref, b3_ref, wd1_ref, bd1_ref, wd2_ref,
                bd2_ref, wd3_ref, bd3_ref, cb_ref, cbm2_ref, cbn_ref,
                iota_ref, recon_ref, idx_ref, loss_ref):
    cb = cb_ref[...]

    def _half(sl):
        x = x_ref[sl, :]
        a = a_ref[sl, :]
        xa = jnp.concatenate([x, a], axis=1)
        h = jnp.maximum(_dot(xa, w1_ref[...]) + b1_ref[...], 0.0)
        h = jnp.maximum(_dot(h, w2_ref[...]) + b2_ref[...], 0.0)
        z_e = _dot(h, w3_ref[...]) + b3_ref[...]

        dist = jax.lax.dot_general(z_e, cbm2_ref[...], (((1,), (1,)), ((), ())),
                                   preferred_element_type=jnp.float32) + cbn_ref[...]

        dmin = jnp.min(dist, axis=1, keepdims=True)
        iota = iota_ref[...]
        idxf = jnp.min(jnp.where(dist == dmin, iota, float(NUM_CODES)), axis=1)
        idx_ref[sl, :] = idxf.astype(jnp.int32)[:, None]

        lpart = jnp.sum(z_e * z_e) + jnp.sum(dmin)

        onehot = (iota == idxf[:, None]).astype(jnp.float32)
        z_q = _dot(onehot, cb)

        za = jnp.concatenate([z_q, a], axis=1)
        h2 = jnp.maximum(_dot(za, wd1_ref[...]) + bd1_ref[...], 0.0)
        h2 = jnp.maximum(_dot(h2, wd2_ref[...]) + bd2_ref[...], 0.0)
        recon_ref[sl, :] = _dot(h2, wd3_ref[...]) + bd3_ref[...]
        return lpart

    l0 = _half(slice(0, BB))
    loss_ref[...] = jnp.reshape(l0, (1, 1, 1))


def kernel(x, archetype_onehot, W1, b1, W2, b2, W3, b3,
           Wd1, bd1, Wd2, bd2, Wd3, bd3, codebook):
    b1r = b1[None, :]
    b2r = b2[None, :]
    b3r = b3[None, :]
    bd1r = bd1[None, :]
    bd2r = bd2[None, :]
    bd3r = bd3[None, :]
    cbn = jnp.sum(codebook ** 2, axis=1)[None, :]
    cbm2 = -2.0 * codebook
    iota_row = jnp.arange(NUM_CODES, dtype=jnp.float32)[None, :]

    grid = (B // BB,)
    row = lambda i: (i, 0)
    rep = lambda i: (0, 0)

    def wspec(arr):
        return pl.BlockSpec(arr.shape, rep)

    recon, idx2d, loss = pl.pallas_call(
        _vqvae_body,
        grid=grid,
        in_specs=[
            pl.BlockSpec((BB, SLOT_DIM), row),
            pl.BlockSpec((BB, NUM_ARCHETYPES), row),
            wspec(W1), wspec(b1r),
            wspec(W2), wspec(b2r),
            wspec(W3), wspec(b3r),
            wspec(Wd1), wspec(bd1r),
            wspec(Wd2), wspec(bd2r),
            wspec(Wd3), wspec(bd3r),
            wspec(codebook), wspec(cbm2), wspec(cbn), wspec(iota_row),
        ],
        out_specs=[
            pl.BlockSpec((BB, SLOT_DIM), row),
            pl.BlockSpec((BB, 1), row),
            pl.BlockSpec((1, 1, 1), lambda i: (i, 0, 0)),
        ],
        out_shape=[
            jax.ShapeDtypeStruct((B, SLOT_DIM), jnp.float32),
            jax.ShapeDtypeStruct((B, 1), jnp.int32),
            jax.ShapeDtypeStruct((B // BB, 1, 1), jnp.float32),
        ],
        compiler_params=pltpu.CompilerParams(
            dimension_semantics=("parallel",),
        ),
    )(x, archetype_onehot, W1, b1r, W2, b2r, W3, b3r,
      Wd1, bd1r, Wd2, bd2r, Wd3, bd3r, codebook, cbm2, cbn, iota_row)

    indices = idx2d[:, 0]
    vq_loss = (COMMIT_COST / (B * CODE_DIM)) * jnp.sum(loss)
    return (recon, indices, vq_loss)


# speedup vs baseline: 1.0203x; 1.0203x over previous
import jax
import jax.numpy as jnp
from jax.experimental import pallas as pl
from jax.experimental.pallas import tpu as pltpu

SLOT_DIM = 142
NUM_ARCHETYPES = 19
HIDDEN_DIM = 256
CODE_DIM = 64
NUM_CODES = 512
COMMIT_COST = 0.25
B = 16384
BB = 4096


def _dot(a, b):
    return jax.lax.dot_general(a, b, (((1,), (0,)), ((), ())),
                               preferred_element_type=jnp.float32)


def _vqvae_body(x_ref, a_ref, w1_ref, b1_ref, w2_ref, b2_ref,
                w3_ref, b3_ref, wd1_ref, bd1_ref, wd2_ref,
                bd2_ref, wd3_ref, bd3_ref, cb_ref, cbm2_ref, cbn_ref,
                iota_ref, recon_ref, idx_ref, loss_ref):
    cb = cb_ref[...]

    def _half(sl):
        x = x_ref[sl, :]
        a = a_ref[sl, :]
        xa = jnp.concatenate([x, a], axis=1)
        h = jnp.maximum(_dot(xa, w1_ref[...]) + b1_ref[...], 0.0)
        h = jnp.maximum(_dot(h, w2_ref[...]) + b2_ref[...], 0.0)
        z_e = _dot(h, w3_ref[...]) + b3_ref[...]

        dist = jax.lax.dot_general(z_e, cbm2_ref[...], (((1,), (1,)), ((), ())),
                                   preferred_element_type=jnp.float32) + cbn_ref[...]

        dmin = jnp.min(dist, axis=1, keepdims=True)
        iota = iota_ref[...]
        idxf = jnp.min(jnp.where(dist == dmin, iota, float(NUM_CODES)), axis=1)
        idx_ref[sl, :] = idxf.astype(jnp.int32)[:, None]

        lpart = jnp.sum(z_e * z_e) + jnp.sum(dmin)

        onehot = (iota == idxf[:, None]).astype(jnp.float32)
        z_q = _dot(onehot, cb)

        za = jnp.concatenate([z_q, a], axis=1)
        h2 = jnp.maximum(_dot(za, wd1_ref[...]) + bd1_ref[...], 0.0)
        h2 = jnp.maximum(_dot(h2, wd2_ref[...]) + bd2_ref[...], 0.0)
        recon_ref[sl, :] = _dot(h2, wd3_ref[...]) + bd3_ref[...]
        return lpart

    l0 = _half(slice(0, BB))
    loss_ref[...] = jnp.reshape(l0, (1, 1, 1))


def kernel(x, archetype_onehot, W1, b1, W2, b2, W3, b3,
           Wd1, bd1, Wd2, bd2, Wd3, bd3, codebook):
    b1r = b1[None, :]
    b2r = b2[None, :]
    b3r = b3[None, :]
    bd1r = bd1[None, :]
    bd2r = bd2[None, :]
    bd3r = bd3[None, :]
    cbn = jnp.sum(codebook ** 2, axis=1)[None, :]
    cbm2 = -2.0 * codebook
    iota_row = jnp.arange(NUM_CODES, dtype=jnp.float32)[None, :]

    grid = (B // BB,)
    row = lambda i: (i, 0)
    rep = lambda i: (0, 0)

    def wspec(arr):
        return pl.BlockSpec(arr.shape, rep)

    recon, idx2d, loss = pl.pallas_call(
        _vqvae_body,
        grid=grid,
        in_specs=[
            pl.BlockSpec((BB, SLOT_DIM), row),
            pl.BlockSpec((BB, NUM_ARCHETYPES), row),
            wspec(W1), wspec(b1r),
            wspec(W2), wspec(b2r),
            wspec(W3), wspec(b3r),
            wspec(Wd1), wspec(bd1r),
            wspec(Wd2), wspec(bd2r),
            wspec(Wd3), wspec(bd3r),
            wspec(codebook), wspec(cbm2), wspec(cbn), wspec(iota_row),
        ],
        out_specs=[
            pl.BlockSpec((BB, SLOT_DIM), row),
            pl.BlockSpec((BB, 1), row),
            pl.BlockSpec((1, 1, 1), lambda i: (i, 0, 0)),
        ],
        out_shape=[
            jax.ShapeDtypeStruct((B, SLOT_DIM), jnp.float32),
            jax.ShapeDtypeStruct((B, 1), jnp.int32),
            jax.ShapeDtypeStruct((B // BB, 1, 1), jnp.float32),
        ],
        compiler_params=pltpu.CompilerParams(
            dimension_semantics=("parallel",),
        ),
    )(x, archetype_onehot, W1, b1r, W2, b2r, W3, b3r,
      Wd1, bd1r, Wd2, bd2r, Wd3, bd3r, codebook, cbm2, cbn, iota_row)

    indices = idx2d[:, 0]
    vq_loss = (COMMIT_COST / (B * CODE_DIM)) * jnp.sum(loss)
    return (recon, indices, vq_loss)
